# baseline (speedup 1.0000x reference)
import jax
import jax.numpy as jnp
from jax.experimental import pallas as pl
from jax.experimental.pallas import tpu as pltpu

_K = 8
_N = 16384
_D = 2048
_R = 512
_B = 6
_C = _N // _R


def _body(x_vmem, x_smem, y_hbm, out_hbm, buf, in_sems, out_sems):
    sub = _R // 128

    def in_cp(i):
        sl = pl.ds(i * _R, _R)
        return pltpu.make_async_copy(y_hbm.at[sl, :], buf.at[i % _B],
                                     in_sems.at[i % _B])

    def out_cp(i):
        sl = pl.ds(i * _R, _R)
        return pltpu.make_async_copy(buf.at[i % _B], out_hbm.at[sl, :],
                                     out_sems.at[i % _B])

    lookahead = _B // 2
    for i in range(lookahead):
        in_cp(i).start()

    for i in range(_C):
        in_cp(i).wait()

        labels = x_vmem[pl.ds(i * sub, sub), :]
        n_bad = jnp.sum(((labels < 0) | (labels >= _K)).astype(jnp.int32))

        @pl.when(n_bad > 0)
        def _fixup(i=i):
            def zero_bad_row(r, _):
                lab = x_smem[i * _R + r]

                @pl.when((lab < 0) | (lab >= _K))
                def _z():
                    buf[i % _B, pl.ds(r, 1), :] = jnp.zeros((1, _D),
                                                            jnp.float32)
                return _
            jax.lax.fori_loop(0, _R, zero_bad_row, 0)

        out_cp(i).start()

        j = i + lookahead
        if j < _C:
            if j >= _B:
                out_cp(j - _B).wait()
            in_cp(j).start()

    for i in range(_C - _B, _C):
        out_cp(i).wait()


def kernel(x, y):
    n, d = y.shape
    xi = x.astype(jnp.int32)
    x2 = xi.reshape(n // 128, 128)
    return pl.pallas_call(
        _body,
        in_specs=[
            pl.BlockSpec(memory_space=pltpu.MemorySpace.VMEM),
            pl.BlockSpec(memory_space=pltpu.MemorySpace.SMEM),
            pl.BlockSpec(memory_space=pl.ANY),
        ],
        out_specs=pl.BlockSpec(memory_space=pl.ANY),
        out_shape=jax.ShapeDtypeStruct((n, d), y.dtype),
        scratch_shapes=[
            pltpu.VMEM((_B, _R, _D), jnp.float32),
            pltpu.SemaphoreType.DMA((_B,)),
            pltpu.SemaphoreType.DMA((_B,)),
        ],
    )(x2, xi, y)


# speedup vs baseline: 1.0031x; 1.0031x over previous
import jax
import jax.numpy as jnp
from jax.experimental import pallas as pl
from jax.experimental.pallas import tpu as pltpu

_K = 8
_N = 16384
_D = 2048
_R = 1024
_B = 4
_C = _N // _R


def _body(x_vmem, x_smem, y_hbm, out_hbm, buf, in_sems, out_sems):
    sub = _R // 128

    def in_cp(i):
        sl = pl.ds(i * _R, _R)
        return pltpu.make_async_copy(y_hbm.at[sl, :], buf.at[i % _B],
                                     in_sems.at[i % _B])

    def out_cp(i):
        sl = pl.ds(i * _R, _R)
        return pltpu.make_async_copy(buf.at[i % _B], out_hbm.at[sl, :],
                                     out_sems.at[i % _B])

    lookahead = _B // 2
    for i in range(lookahead):
        in_cp(i).start()

    for i in range(_C):
        in_cp(i).wait()

        labels = x_vmem[pl.ds(i * sub, sub), :]
        n_bad = jnp.sum(((labels < 0) | (labels >= _K)).astype(jnp.int32))

        @pl.when(n_bad > 0)
        def _fixup(i=i):
            def zero_bad_row(r, _):
                lab = x_smem[i * _R + r]

                @pl.when((lab < 0) | (lab >= _K))
                def _z():
                    buf[i % _B, pl.ds(r, 1), :] = jnp.zeros((1, _D),
                                                            jnp.float32)
                return _
            jax.lax.fori_loop(0, _R, zero_bad_row, 0)

        out_cp(i).start()

        j = i + lookahead
        if j < _C:
            if j >= _B:
                out_cp(j - _B).wait()
            in_cp(j).start()

    for i in range(_C - _B, _C):
        out_cp(i).wait()


def kernel(x, y):
    n, d = y.shape
    xi = x.astype(jnp.int32)
    x2 = xi.reshape(n // 128, 128)
    return pl.pallas_call(
        _body,
        in_specs=[
            pl.BlockSpec(memory_space=pltpu.MemorySpace.VMEM),
            pl.BlockSpec(memory_space=pltpu.MemorySpace.SMEM),
            pl.BlockSpec(memory_space=pl.ANY),
        ],
        out_specs=pl.BlockSpec(memory_space=pl.ANY),
        out_shape=jax.ShapeDtypeStruct((n, d), y.dtype),
        scratch_shapes=[
            pltpu.VMEM((_B, _R, _D), jnp.float32),
            pltpu.SemaphoreType.DMA((_B,)),
            pltpu.SemaphoreType.DMA((_B,)),
        ],
    )(x2, xi, y)


# speedup vs baseline: 1.0170x; 1.0138x over previous
import jax
import jax.numpy as jnp
from jax.experimental import pallas as pl
from jax.experimental.pallas import tpu as pltpu

_K = 8
_R = 2048
_W = 1024


def _body(x_vmem, x_smem, y_ref, out_ref):
    b = pl.program_id(0)
    sub = _R // 128
    labels = x_vmem[pl.ds(b * sub, sub), :]
    n_bad = jnp.sum(((labels < 0) | (labels >= _K)).astype(jnp.int32))

    out_ref[:, :] = y_ref[:, :]

    @pl.when(n_bad > 0)
    def _fixup():
        def zero_bad_row(i, _):
            lab = x_smem[b * _R + i]

            @pl.when((lab < 0) | (lab >= _K))
            def _z():
                out_ref[pl.ds(i, 1), :] = jnp.zeros((1, _W), out_ref.dtype)
            return _
        jax.lax.fori_loop(0, _R, zero_bad_row, 0)


def kernel(x, y):
    n, d = y.shape
    xi = x.astype(jnp.int32)
    x2 = xi.reshape(n // 128, 128)
    return pl.pallas_call(
        _body,
        grid=(n // _R, d // _W),
        in_specs=[
            pl.BlockSpec((n // 128, 128), lambda i, j: (0, 0)),
            pl.BlockSpec(memory_space=pltpu.MemorySpace.SMEM),
            pl.BlockSpec((_R, _W), lambda i, j: (i, j)),
        ],
        out_specs=pl.BlockSpec((_R, _W), lambda i, j: (i, j)),
        out_shape=jax.ShapeDtypeStruct((n, d), y.dtype),
        compiler_params=pltpu.CompilerParams(
            dimension_semantics=("arbitrary", "arbitrary"),
        ),
    )(x2, xi, y)
